# Initial kernel scaffold; baseline (speedup 1.0000x reference)
import jax
import jax.numpy as jnp
from jax.experimental import pallas as pl


def kernel(pred_logits, pred_boxes, pred_keypoints, orig_target_sizes):
    raise NotImplementedError("write your pallas kernel here")


# speedup vs baseline: 1.0035x; 1.0035x over previous
import jax
import jax.numpy as jnp
from jax.experimental import pallas as pl

NUM_CLASSES = 80
TOPK = 300


def kernel(pred_logits, pred_boxes, pred_keypoints, orig_target_sizes):
    B = pred_logits.shape[0]
    flat = pred_logits.reshape(B, -1)
    top_logits, index = jax.lax.top_k(flat, TOPK)
    top_scores = jax.nn.sigmoid(top_logits)
    labels = index % NUM_CLASSES
    qidx = index // NUM_CLASSES
    sizes = orig_target_sizes.astype(jnp.float32)
    boxes_sel = jnp.take_along_axis(pred_boxes, qidx[:, :, None], axis=1)
    cx, cy, w, h = (boxes_sel[..., 0], boxes_sel[..., 1],
                    boxes_sel[..., 2], boxes_sel[..., 3])
    xyxy = jnp.stack([cx - 0.5 * w, cy - 0.5 * h, cx + 0.5 * w, cy + 0.5 * h],
                     axis=-1)
    boxes = xyxy * jnp.tile(sizes, (1, 2))[:, None, :]
    kpts_sel = jnp.take_along_axis(pred_keypoints, qidx[:, :, None, None],
                                   axis=1)
    kpts = kpts_sel * sizes[:, None, None, :]
    return (labels, boxes, top_scores, kpts)


# speedup vs baseline: 1.3555x; 1.3508x over previous
import functools

import jax
import jax.numpy as jnp
from jax import lax
from jax.experimental import pallas as pl
from jax.experimental.pallas import tpu as pltpu
from jax.experimental.pallas import tpu_sc as plsc

NCLS = 80
NQ = 5000
K = 300
BATCH = 16
NFLAT = NQ * NCLS
W = 16000
NWIN = NFLAT // W
CHW = W // 16
BINS = 2048
CEQ = 4096
GTCAP = 640
SELP = 304
MASK31 = 0x7FFFFFFF


def _i32(x):
    return jnp.int32(x)


def _shrl(v, n):
    n_arr = jnp.broadcast_to(jnp.asarray(n, jnp.int32), jnp.shape(v))
    return lax.shift_right_logical(v, n_arr)


def _body(lg, boxes1d, kpts1d, sizes_hbm,
          lab_hbm, box_hbm, sco_hbm, kpt_hbm,
          wbuf, hist, totals,
          gt_key, gt_idx, eq_key, eq_idx, eq2_idx,
          sel_key, sel_idx,
          out_lab_v, out_sco_v, qrank_v, qabs_v,
          bidx_v, kidx_v, boxgf, boxof, kpgf, sizes_v,
          sem0, sem1, semg, semg2):
    cid = lax.axis_index("c")
    b = lax.axis_index("s")
    lane = lax.iota(jnp.int32, 16)
    ones16 = jnp.ones((16,), jnp.int32)
    zeros16 = jnp.zeros((16,), jnp.int32)
    MIN = _i32(-2**31)
    lane_off = lane * BINS
    b_base = b * NFLAT

    def keyof(x16):
        bb = lax.bitcast_convert_type(x16, jnp.int32)
        return jnp.where(bb < 0, ~(bb & _i32(MASK31)), bb)

    def zero_hist():
        def zh(i, _):
            hist[pl.ds(i * 16, 16)] = zeros16
            return 0
        lax.fori_loop(0, BINS * 16 // 16, zh, 0)

    def merge_lanes():
        def ml(j, _):
            acc = zeros16
            for l in range(16):
                acc = acc + hist[pl.ds(l * BINS + j * 16, 16)]
            totals[pl.ds(j * 16, 16)] = acc
            return 0
        lax.fori_loop(0, BINS // 16, ml, 0)

    def lane_at(vec, l):
        return jnp.sum(jnp.where(lane == l, vec, 0))

    def boundary_search(rn):
        def bs(jj, c):
            found, T, G, Q, acc = c
            j = BINS // 16 - 1 - jj
            t = totals[pl.ds(j * 16, 16)]
            rt = jnp.flip(t, 0)
            cs = plsc.cumsum(rt)
            s_inc = cs + acc
            hitm = s_inc >= rn
            cnt = jnp.sum(hitm.astype(jnp.int32))
            hit = jnp.logical_and(found == 0, cnt > 0)
            lstar = 16 - cnt
            csl = lane_at(cs, lstar)
            rtl = lane_at(rt, lstar)
            T = jnp.where(hit, j * 16 + 15 - lstar, T)
            G = jnp.where(hit, acc + csl - rtl, G)
            Q = jnp.where(hit, rtl, Q)
            found = jnp.where(hit, 1, found)
            acc = acc + jnp.sum(t)
            return (found, T, G, Q, acc)
        init = (_i32(0), _i32(0), _i32(0), _i32(0), _i32(0))
        _, T, G, Q, _ = lax.fori_loop(0, BINS // 16, bs, init)
        return T, G, Q

    def stream_windows(process, carry):
        pltpu.async_copy(lg.at[pl.ds(b_base, W)], wbuf.at[pl.ds(0, W)], sem0)

        def outer(w2, c):
            wA = 2 * w2
            pltpu.async_copy(lg.at[pl.ds(b_base + (wA + 1) * W, W)],
                             wbuf.at[pl.ds(W, W)], sem1)
            pltpu.make_async_copy(lg.at[pl.ds(b_base + wA * W, W)],
                                  wbuf.at[pl.ds(0, W)], sem0).wait()
            c = process(0, wA, c)
            pltpu.async_copy(lg.at[pl.ds(b_base + (wA + 2) * W, W)],
                             wbuf.at[pl.ds(0, W)], sem0)
            pltpu.make_async_copy(lg.at[pl.ds(b_base + (wA + 1) * W, W)],
                                  wbuf.at[pl.ds(W, W)], sem1).wait()
            c = process(W, wA + 1, c)
            return c
        carry = lax.fori_loop(0, (NWIN - 1) // 2, outer, carry)
        pltpu.make_async_copy(lg.at[pl.ds(b_base + (NWIN - 1) * W, W)],
                              wbuf.at[pl.ds(0, W)], sem0).wait()
        carry = process(0, NWIN - 1, carry)
        return carry

    @pl.when(cid == 0)
    def _work():
        pltpu.sync_copy(sizes_hbm.at[b], sizes_v)

        zero_hist()

        def hist_chunk(soff, w, c):
            def inner(i, _):
                x = wbuf[pl.ds(soff + i * 16, 16)]
                ku = keyof(x) ^ MIN
                dig = _shrl(ku, 21)
                plsc.addupdate_scatter(hist, [lane_off + dig], ones16)
                return 0
            lax.fori_loop(0, CHW, inner, 0)
            return c
        stream_windows(hist_chunk, 0)
        merge_lanes()
        T, G, Q = boundary_search(_i32(K))
        shift = _i32(21)
        pref = T
        rn = _i32(K) - G

        def fb_cond(st):
            shift, pref, rn, Q = st
            return jnp.logical_and(Q > CEQ, shift > 0)

        def fb_body(st):
            shift, pref, rn, Q = st
            new_w = jnp.minimum(_i32(11), shift)
            new_shift = shift - new_w
            wmask = (_i32(1) << new_w) - 1
            zero_hist()

            def fw(w, _):
                pltpu.sync_copy(lg.at[pl.ds(b_base + w * W, W)],
                                wbuf.at[pl.ds(0, W)])

                def inner(i, _):
                    x = wbuf[pl.ds(0 + i * 16, 16)]
                    ku = keyof(x) ^ MIN
                    pm = _shrl(ku, shift) == pref
                    dig = _shrl(ku, new_shift) & wmask
                    plsc.addupdate_scatter(hist, [lane_off + dig], ones16,
                                           mask=pm)
                    return 0
                lax.fori_loop(0, CHW, inner, 0)
                return 0
            lax.fori_loop(0, NWIN, fw, 0)
            merge_lanes()
            T, G, Q = boundary_search(rn)
            return ((new_shift, (pref << new_w) | T, rn - G, Q))

        shift, pref, rn, Q = lax.while_loop(
            fb_cond, fb_body, (shift, pref, rn, Q))

        pref_b = pref ^ MIN

        def compact_chunk(soff, w, c):
            def inner(i, c2):
                off_gt, off_eq = c2
                x = wbuf[pl.ds(soff + i * 16, 16)]
                ks = keyof(x)
                ku = ks ^ MIN
                top = _shrl(ku, shift)
                m_gt = (top ^ MIN) > pref_b
                m_eq = top == pref
                n_gt = jnp.sum(m_gt.astype(jnp.int32))
                n_eq = jnp.sum(m_eq.astype(jnp.int32))
                fidx = w * W + i * 16 + lane

                @pl.when(n_gt + n_eq > 0)
                def _st():
                    plsc.store_compressed(gt_key.at[pl.ds(off_gt, 16)], ks,
                                          mask=m_gt)
                    plsc.store_compressed(gt_idx.at[pl.ds(off_gt, 16)], fidx,
                                          mask=m_gt)

                    @pl.when(off_eq < CEQ)
                    def _se():
                        plsc.store_compressed(eq_key.at[pl.ds(off_eq, 16)],
                                              ks, mask=m_eq)
                        plsc.store_compressed(eq_idx.at[pl.ds(off_eq, 16)],
                                              fidx, mask=m_eq)
                off_gt = off_gt + n_gt
                off_eq = jnp.where(off_eq < CEQ, off_eq + n_eq, off_eq)
                return (off_gt, off_eq)

            return lax.fori_loop(0, CHW, inner, c)

        off_gt, off_eq = stream_windows(compact_chunk, (_i32(0), _i32(0)))
        Qs = off_eq
        nch_eq = (Qs + 15) // 16

        def local_hist(shift_p, pref_p, new_w, new_shift):
            zero_hist()
            wmask = (_i32(1) << new_w) - 1

            def lh(ci, _):
                kk = eq_key[pl.ds(ci * 16, 16)]
                valid = (ci * 16 + lane) < Qs
                ku2 = kk ^ MIN
                pm = jnp.logical_and(valid, _shrl(ku2, shift_p) == pref_p)
                dig = _shrl(ku2, new_shift) & wmask
                plsc.addupdate_scatter(hist, [lane_off + dig], ones16,
                                       mask=pm)
                return 0
            lax.fori_loop(0, nch_eq, lh, 0)
            merge_lanes()

        w1 = jnp.minimum(_i32(11), shift)
        sh1 = shift - w1
        local_hist(shift, pref, w1, sh1)
        T, G, Q = boundary_search(rn)
        pref = (pref << w1) | T
        rn = rn - G
        shift = sh1

        w2 = shift
        local_hist(shift, pref, w2, _i32(0))
        T, G, Q = boundary_search(rn)
        pref = (pref << w2) | T
        rn = rn - G
        E = rn
        k300_ks = pref ^ MIN

        def lsel(ci, c2):
            off_g, off_e2 = c2
            kk = eq_key[pl.ds(ci * 16, 16)]
            ii = eq_idx[pl.ds(ci * 16, 16)]
            valid = (ci * 16 + lane) < Qs
            m_g2 = jnp.logical_and(valid, kk > k300_ks)
            m_e2 = jnp.logical_and(valid, kk == k300_ks)
            n_g2 = jnp.sum(m_g2.astype(jnp.int32))
            n_e2 = jnp.sum(m_e2.astype(jnp.int32))

            @pl.when(n_g2 + n_e2 > 0)
            def _st():
                plsc.store_compressed(gt_key.at[pl.ds(off_g, 16)], kk,
                                      mask=m_g2)
                plsc.store_compressed(gt_idx.at[pl.ds(off_g, 16)], ii,
                                      mask=m_g2)

                @pl.when(off_e2 < SELP)
                def _se():
                    plsc.store_compressed(eq2_idx.at[pl.ds(off_e2, 16)], ii,
                                          mask=m_e2)
            off_g = off_g + n_g2
            off_e2 = jnp.where(off_e2 < SELP, off_e2 + n_e2, off_e2)
            return (off_g, off_e2)

        gp, _ = lax.fori_loop(0, nch_eq, lsel, (off_gt, _i32(0)))

        for c in range(SELP // 16):
            pos = c * 16 + lane
            in_gt = pos < gp
            gk = plsc.load_gather(gt_key, [jnp.minimum(pos, GTCAP - 1)],
                                  mask=in_gt)
            gi = plsc.load_gather(gt_idx, [jnp.minimum(pos, GTCAP - 1)],
                                  mask=in_gt)
            in_eq = jnp.logical_and(jnp.logical_not(in_gt), pos < K)
            ep = jnp.clip(pos - gp, 0, SELP + 15)
            ei = plsc.load_gather(eq2_idx, [ep], mask=in_eq)
            keyc = jnp.where(in_gt, gk, jnp.where(in_eq, k300_ks, MIN))
            idxc = jnp.where(in_gt, gi,
                             jnp.where(in_eq, ei, _i32(MASK31)))
            sel_key[pl.ds(c * 16, 16)] = keyc
            sel_idx[pl.ds(c * 16, 16)] = idxc

        def rank_j(j, ranks):
            jc = (j // 16) * 16
            lj = j - jc
            kv = sel_key[pl.ds(jc, 16)]
            iv = sel_idx[pl.ds(jc, 16)]
            kj = lane_at(kv, lj)
            ij = lane_at(iv, lj)
            new = []
            for c in range(SELP // 16):
                ki = sel_key[pl.ds(c * 16, 16)]
                ii2 = sel_idx[pl.ds(c * 16, 16)]
                beat = jnp.logical_or(
                    kj > ki,
                    jnp.logical_and(kj == ki, ij < ii2))
                new.append(ranks[c] + beat.astype(jnp.int32))
            return tuple(new)

        ranks0 = tuple(jnp.zeros((16,), jnp.int32)
                       for _ in range(SELP // 16))
        ranks = lax.fori_loop(0, SELP, rank_j, ranks0)

        out_lab_v[pl.ds(288, 16)] = zeros16
        out_sco_v[pl.ds(288, 16)] = jnp.zeros((16,), jnp.float32)
        qrank_v[pl.ds(288, 16)] = zeros16
        for c in range(SELP // 16):
            rk = ranks[c]
            valid = rk < K
            ksv = sel_key[pl.ds(c * 16, 16)]
            bb = jnp.where(ksv >= 0, ksv, (~ksv) | MIN)
            val = lax.bitcast_convert_type(bb, jnp.float32)
            score = 1.0 / (1.0 + jnp.exp(-val))
            fi = sel_idx[pl.ds(c * 16, 16)]
            q = fi // NCLS
            labv = fi - q * NCLS
            plsc.store_scatter(out_sco_v, [rk], score, mask=valid)
            plsc.store_scatter(out_lab_v, [rk], labv, mask=valid)
            plsc.store_scatter(qrank_v, [rk], q, mask=valid)

        for c in range(SELP // 16):
            qv = qrank_v[pl.ds(c * 16, 16)]
            qabs_v[pl.ds(c * 16, 16)] = qv + b * NQ

        for c in range(SELP * 4 // 16):
            rr = c * 4 + lax.shift_right_logical(lane, jnp.full((16,), 2,
                                                                jnp.int32))
            qv4 = plsc.load_gather(qabs_v, [rr])
            bidx_v[pl.ds(c * 16, 16)] = qv4 * 4 + (lane & 3)

        def kidx(ci, _):
            flat = ci * 16 + lane
            r = flat // 34
            col = flat - r * 34
            qv = plsc.load_gather(qabs_v, [r])
            kidx_v[pl.ds(ci * 16, 16)] = qv * 34 + col
            return 0
        lax.fori_loop(0, SELP * 34 // 16, kidx, 0)

        cb = pltpu.async_copy(boxes1d.at[bidx_v], boxgf, semg)
        ck = pltpu.async_copy(kpts1d.at[kidx_v], kpgf, semg2)
        cb.wait()
        ck.wait()

        sz = sizes_v[pl.ds(0, 16)]
        s0 = jnp.sum(jnp.where(lane == 0, sz, 0.0))
        s1 = jnp.sum(jnp.where(lane == 1, sz, 0.0))

        for c in range(SELP // 16):
            r4 = (c * 16 + lane) * 4
            cx = plsc.load_gather(boxgf, [r4])
            cy = plsc.load_gather(boxgf, [r4 + 1])
            wv = plsc.load_gather(boxgf, [r4 + 2])
            hv = plsc.load_gather(boxgf, [r4 + 3])
            hw = wv * 0.5
            hh = hv * 0.5
            plsc.store_scatter(boxof, [r4], (cx - hw) * s0)
            plsc.store_scatter(boxof, [r4 + 1], (cy - hh) * s1)
            plsc.store_scatter(boxof, [r4 + 2], (cx + hw) * s0)
            plsc.store_scatter(boxof, [r4 + 3], (cy + hh) * s1)

        whk = jnp.where((lane & 1) == 0, s0, s1)

        def kp(ci, _):
            kpgf[pl.ds(ci * 16, 16)] = kpgf[pl.ds(ci * 16, 16)] * whk
            return 0
        lax.fori_loop(0, SELP * 34 // 16, kp, 0)

        pltpu.sync_copy(out_lab_v, lab_hbm.at[b])
        pltpu.sync_copy(out_sco_v, sco_hbm.at[b])
        pltpu.sync_copy(boxof.at[pl.ds(0, K * 4)], box_hbm.at[b])
        pltpu.sync_copy(kpgf.at[pl.ds(0, K * 34)], kpt_hbm.at[b])


@jax.jit
def _run(lg, boxes2d, kpts2d, sizes_pad):
    mesh = plsc.VectorSubcoreMesh(core_axis_name="c", subcore_axis_name="s")
    fn = pl.kernel(
        _body,
        out_type=[
            jax.ShapeDtypeStruct((BATCH, SELP), jnp.int32),
            jax.ShapeDtypeStruct((BATCH, K * 4), jnp.float32),
            jax.ShapeDtypeStruct((BATCH, SELP), jnp.float32),
            jax.ShapeDtypeStruct((BATCH, K * 34), jnp.float32),
        ],
        mesh=mesh,
        compiler_params=pltpu.CompilerParams(needs_layout_passes=False,
                                             use_tc_tiling_on_sc=False),
        scratch_types=[
            pltpu.VMEM((2 * W,), jnp.float32),
            pltpu.VMEM((BINS * 16,), jnp.int32),
            pltpu.VMEM((BINS,), jnp.int32),
            pltpu.VMEM((GTCAP,), jnp.int32),
            pltpu.VMEM((GTCAP,), jnp.int32),
            pltpu.VMEM((CEQ + 16,), jnp.int32),
            pltpu.VMEM((CEQ + 16,), jnp.int32),
            pltpu.VMEM((SELP + 16,), jnp.int32),
            pltpu.VMEM((SELP,), jnp.int32),
            pltpu.VMEM((SELP,), jnp.int32),
            pltpu.VMEM((SELP,), jnp.int32),
            pltpu.VMEM((SELP,), jnp.float32),
            pltpu.VMEM((SELP,), jnp.int32),
            pltpu.VMEM((SELP,), jnp.int32),
            pltpu.VMEM((SELP * 4,), jnp.int32),
            pltpu.VMEM((SELP * 34,), jnp.int32),
            pltpu.VMEM((SELP * 4,), jnp.float32),
            pltpu.VMEM((SELP * 4,), jnp.float32),
            pltpu.VMEM((SELP * 34,), jnp.float32),
            pltpu.VMEM((16,), jnp.float32),
            pltpu.SemaphoreType.DMA,
            pltpu.SemaphoreType.DMA,
            pltpu.SemaphoreType.DMA,
            pltpu.SemaphoreType.DMA,
        ],
    )
    return fn(lg, boxes2d, kpts2d, sizes_pad)


def kernel(pred_logits, pred_boxes, pred_keypoints, orig_target_sizes):
    lg = pred_logits.reshape(-1)
    boxes1d = pred_boxes.reshape(-1)
    kpts1d = pred_keypoints.reshape(-1)
    sizes_pad = jnp.pad(orig_target_sizes.astype(jnp.float32),
                        ((0, 0), (0, 14)))
    lab_p, box_p, sco_p, kpt_p = _run(lg, boxes1d, kpts1d, sizes_pad)
    labels = lab_p[:, :K]
    scores = sco_p[:, :K]
    boxes = box_p.reshape(BATCH, K, 4)
    kpts = kpt_p.reshape(BATCH, K, 17, 2)
    return (labels, boxes, scores, kpts)


# speedup vs baseline: 1.5525x; 1.1453x over previous
import functools

import jax
import jax.numpy as jnp
from jax import lax
from jax.experimental import pallas as pl
from jax.experimental.pallas import tpu as pltpu
from jax.experimental.pallas import tpu_sc as plsc

NCLS = 80
NQ = 5000
K = 300
BATCH = 16
NFLAT = NQ * NCLS
W = 16000
NWIN = NFLAT // W
CHW = W // 16
BINS = 2048
CEQ = 4096
GTCAP = 640
SELP = 304
MASK31 = 0x7FFFFFFF


def _i32(x):
    return jnp.int32(x)


def _shrl(v, n):
    n_arr = jnp.broadcast_to(jnp.asarray(n, jnp.int32), jnp.shape(v))
    return lax.shift_right_logical(v, n_arr)


def _body(lg, boxes1d, kpts1d, sizes_hbm,
          lab_hbm, box_hbm, sco_hbm, kpt_hbm,
          wbuf, hist, totals,
          gt_key, gt_idx, eq_key, eq_idx, eq2_idx,
          sel_key, sel_idx,
          out_lab_v, out_sco_v, qrank_v, qabs_v,
          bidx_v, kidx_v, boxgf, boxof, kpgf, sizes_v,
          sem0, sem1, semg, semg2):
    cid = lax.axis_index("c")
    b = lax.axis_index("s")
    lane = lax.iota(jnp.int32, 16)
    ones16 = jnp.ones((16,), jnp.int32)
    zeros16 = jnp.zeros((16,), jnp.int32)
    MIN = _i32(-2**31)
    lane_off = lane * BINS
    b_base = b * NFLAT

    def keyof(x16):
        bb = lax.bitcast_convert_type(x16, jnp.int32)
        return jnp.where(bb < 0, ~(bb & _i32(MASK31)), bb)

    def zero_hist():
        def zh(i, _):
            for u in range(8):
                hist[pl.ds((i * 8 + u) * 16, 16)] = zeros16
            return 0
        lax.fori_loop(0, BINS * 16 // 16 // 8, zh, 0)

    def merge_lanes():
        def ml(j, _):
            acc = zeros16
            for l in range(16):
                acc = acc + hist[pl.ds(l * BINS + j * 16, 16)]
            totals[pl.ds(j * 16, 16)] = acc
            return 0
        lax.fori_loop(0, BINS // 16, ml, 0)

    def lane_at(vec, l):
        return jnp.sum(jnp.where(lane == l, vec, 0))

    def boundary_search(rn):
        def bs(jj, c):
            found, T, G, Q, acc = c
            j = BINS // 16 - 1 - jj
            t = totals[pl.ds(j * 16, 16)]
            rt = jnp.flip(t, 0)
            cs = plsc.cumsum(rt)
            s_inc = cs + acc
            hitm = s_inc >= rn
            cnt = jnp.sum(hitm.astype(jnp.int32))
            hit = jnp.logical_and(found == 0, cnt > 0)
            lstar = 16 - cnt
            csl = lane_at(cs, lstar)
            rtl = lane_at(rt, lstar)
            T = jnp.where(hit, j * 16 + 15 - lstar, T)
            G = jnp.where(hit, acc + csl - rtl, G)
            Q = jnp.where(hit, rtl, Q)
            found = jnp.where(hit, 1, found)
            acc = acc + jnp.sum(t)
            return (found, T, G, Q, acc)
        init = (_i32(0), _i32(0), _i32(0), _i32(0), _i32(0))
        _, T, G, Q, _ = lax.fori_loop(0, BINS // 16, bs, init)
        return T, G, Q

    def stream_windows(process, carry):
        pltpu.async_copy(lg.at[pl.ds(b_base, W)], wbuf.at[pl.ds(0, W)], sem0)

        def outer(w2, c):
            wA = 2 * w2
            pltpu.async_copy(lg.at[pl.ds(b_base + (wA + 1) * W, W)],
                             wbuf.at[pl.ds(W, W)], sem1)
            pltpu.make_async_copy(lg.at[pl.ds(b_base + wA * W, W)],
                                  wbuf.at[pl.ds(0, W)], sem0).wait()
            c = process(0, wA, c)
            pltpu.async_copy(lg.at[pl.ds(b_base + (wA + 2) * W, W)],
                             wbuf.at[pl.ds(0, W)], sem0)
            pltpu.make_async_copy(lg.at[pl.ds(b_base + (wA + 1) * W, W)],
                                  wbuf.at[pl.ds(W, W)], sem1).wait()
            c = process(W, wA + 1, c)
            return c
        carry = lax.fori_loop(0, (NWIN - 1) // 2, outer, carry)
        pltpu.make_async_copy(lg.at[pl.ds(b_base + (NWIN - 1) * W, W)],
                              wbuf.at[pl.ds(0, W)], sem0).wait()
        carry = process(0, NWIN - 1, carry)
        return carry

    @pl.when(cid == 0)
    def _work():
        pltpu.sync_copy(sizes_hbm.at[b], sizes_v)

        zero_hist()

        def hist_chunk(soff, w, c):
            def inner(i, _):
                for u in range(8):
                    x = wbuf[pl.ds(soff + (i * 8 + u) * 16, 16)]
                    ku = keyof(x) ^ MIN
                    dig = _shrl(ku, 21)
                    plsc.addupdate_scatter(hist, [lane_off + dig], ones16)
                return 0
            lax.fori_loop(0, CHW // 8, inner, 0)
            return c
        stream_windows(hist_chunk, 0)
        merge_lanes()
        T, G, Q = boundary_search(_i32(K))
        shift = _i32(21)
        pref = T
        rn = _i32(K) - G

        def fb_cond(st):
            shift, pref, rn, Q = st
            return jnp.logical_and(Q > CEQ, shift > 0)

        def fb_body(st):
            shift, pref, rn, Q = st
            new_w = jnp.minimum(_i32(11), shift)
            new_shift = shift - new_w
            wmask = (_i32(1) << new_w) - 1
            zero_hist()

            def fw(w, _):
                pltpu.sync_copy(lg.at[pl.ds(b_base + w * W, W)],
                                wbuf.at[pl.ds(0, W)])

                def inner(i, _):
                    x = wbuf[pl.ds(0 + i * 16, 16)]
                    ku = keyof(x) ^ MIN
                    pm = _shrl(ku, shift) == pref
                    dig = _shrl(ku, new_shift) & wmask
                    plsc.addupdate_scatter(hist, [lane_off + dig], ones16,
                                           mask=pm)
                    return 0
                lax.fori_loop(0, CHW, inner, 0)
                return 0
            lax.fori_loop(0, NWIN, fw, 0)
            merge_lanes()
            T, G, Q = boundary_search(rn)
            return ((new_shift, (pref << new_w) | T, rn - G, Q))

        shift, pref, rn, Q = lax.while_loop(
            fb_cond, fb_body, (shift, pref, rn, Q))

        pref_b = pref ^ MIN

        def compact_chunk(soff, w, c):
            def inner(i, c2):
                vals = []
                anym = None
                for u in range(4):
                    x = wbuf[pl.ds(soff + (i * 4 + u) * 16, 16)]
                    ks = keyof(x)
                    top = _shrl(ks ^ MIN, shift)
                    m_gt = (top ^ MIN) > pref_b
                    m_eq = top == pref
                    vals.append((ks, m_gt, m_eq))
                    m_any = jnp.logical_or(m_gt, m_eq)
                    anym = m_any if anym is None else jnp.logical_or(anym,
                                                                     m_any)
                hit = jnp.sum(anym.astype(jnp.int32))

                def slow(c2):
                    off_gt, off_eq = c2
                    for u, (ks, m_gt, m_eq) in enumerate(vals):
                        fidx = w * W + (i * 4 + u) * 16 + lane
                        n_gt = jnp.sum(m_gt.astype(jnp.int32))
                        n_eq = jnp.sum(m_eq.astype(jnp.int32))
                        plsc.store_compressed(gt_key.at[pl.ds(off_gt, 16)],
                                              ks, mask=m_gt)
                        plsc.store_compressed(gt_idx.at[pl.ds(off_gt, 16)],
                                              fidx, mask=m_gt)

                        @pl.when(off_eq < CEQ)
                        def _se(ks=ks, m_eq=m_eq, fidx=fidx, off_eq=off_eq):
                            plsc.store_compressed(
                                eq_key.at[pl.ds(off_eq, 16)], ks, mask=m_eq)
                            plsc.store_compressed(
                                eq_idx.at[pl.ds(off_eq, 16)], fidx, mask=m_eq)
                        off_gt = off_gt + n_gt
                        off_eq = jnp.where(off_eq < CEQ, off_eq + n_eq,
                                           off_eq)
                    return (off_gt, off_eq)

                return lax.cond(hit > 0, slow, lambda c2: c2, c2)

            return lax.fori_loop(0, CHW // 4, inner, c)

        off_gt, off_eq = stream_windows(compact_chunk, (_i32(0), _i32(0)))
        Qs = off_eq
        nch_eq = (Qs + 15) // 16

        def local_hist(shift_p, pref_p, new_w, new_shift):
            zero_hist()
            wmask = (_i32(1) << new_w) - 1

            def lh(ci, _):
                kk = eq_key[pl.ds(ci * 16, 16)]
                valid = (ci * 16 + lane) < Qs
                ku2 = kk ^ MIN
                pm = jnp.logical_and(valid, _shrl(ku2, shift_p) == pref_p)
                dig = _shrl(ku2, new_shift) & wmask
                plsc.addupdate_scatter(hist, [lane_off + dig], ones16,
                                       mask=pm)
                return 0
            lax.fori_loop(0, nch_eq, lh, 0)
            merge_lanes()

        w1 = jnp.minimum(_i32(11), shift)
        sh1 = shift - w1
        local_hist(shift, pref, w1, sh1)
        T, G, Q = boundary_search(rn)
        pref = (pref << w1) | T
        rn = rn - G
        shift = sh1

        w2 = shift
        local_hist(shift, pref, w2, _i32(0))
        T, G, Q = boundary_search(rn)
        pref = (pref << w2) | T
        rn = rn - G
        E = rn
        k300_ks = pref ^ MIN

        def lsel(ci, c2):
            off_g, off_e2 = c2
            kk = eq_key[pl.ds(ci * 16, 16)]
            ii = eq_idx[pl.ds(ci * 16, 16)]
            valid = (ci * 16 + lane) < Qs
            m_g2 = jnp.logical_and(valid, kk > k300_ks)
            m_e2 = jnp.logical_and(valid, kk == k300_ks)
            n_g2 = jnp.sum(m_g2.astype(jnp.int32))
            n_e2 = jnp.sum(m_e2.astype(jnp.int32))

            @pl.when(n_g2 + n_e2 > 0)
            def _st():
                plsc.store_compressed(gt_key.at[pl.ds(off_g, 16)], kk,
                                      mask=m_g2)
                plsc.store_compressed(gt_idx.at[pl.ds(off_g, 16)], ii,
                                      mask=m_g2)

                @pl.when(off_e2 < SELP)
                def _se():
                    plsc.store_compressed(eq2_idx.at[pl.ds(off_e2, 16)], ii,
                                          mask=m_e2)
            off_g = off_g + n_g2
            off_e2 = jnp.where(off_e2 < SELP, off_e2 + n_e2, off_e2)
            return (off_g, off_e2)

        gp, _ = lax.fori_loop(0, nch_eq, lsel, (off_gt, _i32(0)))

        for c in range(SELP // 16):
            pos = c * 16 + lane
            in_gt = pos < gp
            gk = plsc.load_gather(gt_key, [jnp.minimum(pos, GTCAP - 1)],
                                  mask=in_gt)
            gi = plsc.load_gather(gt_idx, [jnp.minimum(pos, GTCAP - 1)],
                                  mask=in_gt)
            in_eq = jnp.logical_and(jnp.logical_not(in_gt), pos < K)
            ep = jnp.clip(pos - gp, 0, SELP + 15)
            ei = plsc.load_gather(eq2_idx, [ep], mask=in_eq)
            keyc = jnp.where(in_gt, gk, jnp.where(in_eq, k300_ks, MIN))
            idxc = jnp.where(in_gt, gi,
                             jnp.where(in_eq, ei, _i32(MASK31)))
            sel_key[pl.ds(c * 16, 16)] = keyc
            sel_idx[pl.ds(c * 16, 16)] = idxc

        def rank_j(j, ranks):
            jc = (j // 16) * 16
            lj = j - jc
            kv = sel_key[pl.ds(jc, 16)]
            iv = sel_idx[pl.ds(jc, 16)]
            kj = lane_at(kv, lj)
            ij = lane_at(iv, lj)
            new = []
            for c in range(SELP // 16):
                ki = sel_key[pl.ds(c * 16, 16)]
                ii2 = sel_idx[pl.ds(c * 16, 16)]
                beat = jnp.logical_or(
                    kj > ki,
                    jnp.logical_and(kj == ki, ij < ii2))
                new.append(ranks[c] + beat.astype(jnp.int32))
            return tuple(new)

        ranks0 = tuple(jnp.zeros((16,), jnp.int32)
                       for _ in range(SELP // 16))
        ranks = lax.fori_loop(0, SELP, rank_j, ranks0)

        out_lab_v[pl.ds(288, 16)] = zeros16
        out_sco_v[pl.ds(288, 16)] = jnp.zeros((16,), jnp.float32)
        qrank_v[pl.ds(288, 16)] = zeros16
        for c in range(SELP // 16):
            rk = ranks[c]
            valid = rk < K
            ksv = sel_key[pl.ds(c * 16, 16)]
            bb = jnp.where(ksv >= 0, ksv, (~ksv) | MIN)
            val = lax.bitcast_convert_type(bb, jnp.float32)
            score = 1.0 / (1.0 + jnp.exp(-val))
            fi = sel_idx[pl.ds(c * 16, 16)]
            q = fi // NCLS
            labv = fi - q * NCLS
            plsc.store_scatter(out_sco_v, [rk], score, mask=valid)
            plsc.store_scatter(out_lab_v, [rk], labv, mask=valid)
            plsc.store_scatter(qrank_v, [rk], q, mask=valid)

        for c in range(SELP // 16):
            qv = qrank_v[pl.ds(c * 16, 16)]
            qabs_v[pl.ds(c * 16, 16)] = qv + b * NQ

        for c in range(SELP * 4 // 16):
            rr = c * 4 + lax.shift_right_logical(lane, jnp.full((16,), 2,
                                                                jnp.int32))
            qv4 = plsc.load_gather(qabs_v, [rr])
            bidx_v[pl.ds(c * 16, 16)] = qv4 * 4 + (lane & 3)

        def kidx(ci, _):
            flat = ci * 16 + lane
            r = flat // 34
            col = flat - r * 34
            qv = plsc.load_gather(qabs_v, [r])
            kidx_v[pl.ds(ci * 16, 16)] = qv * 34 + col
            return 0
        lax.fori_loop(0, SELP * 34 // 16, kidx, 0)

        cb = pltpu.async_copy(boxes1d.at[bidx_v], boxgf, semg)
        ck = pltpu.async_copy(kpts1d.at[kidx_v], kpgf, semg2)
        cb.wait()
        ck.wait()

        sz = sizes_v[pl.ds(0, 16)]
        s0 = jnp.sum(jnp.where(lane == 0, sz, 0.0))
        s1 = jnp.sum(jnp.where(lane == 1, sz, 0.0))

        for c in range(SELP // 16):
            r4 = (c * 16 + lane) * 4
            cx = plsc.load_gather(boxgf, [r4])
            cy = plsc.load_gather(boxgf, [r4 + 1])
            wv = plsc.load_gather(boxgf, [r4 + 2])
            hv = plsc.load_gather(boxgf, [r4 + 3])
            hw = wv * 0.5
            hh = hv * 0.5
            plsc.store_scatter(boxof, [r4], (cx - hw) * s0)
            plsc.store_scatter(boxof, [r4 + 1], (cy - hh) * s1)
            plsc.store_scatter(boxof, [r4 + 2], (cx + hw) * s0)
            plsc.store_scatter(boxof, [r4 + 3], (cy + hh) * s1)

        whk = jnp.where((lane & 1) == 0, s0, s1)

        def kp(ci, _):
            kpgf[pl.ds(ci * 16, 16)] = kpgf[pl.ds(ci * 16, 16)] * whk
            return 0
        lax.fori_loop(0, SELP * 34 // 16, kp, 0)

        pltpu.sync_copy(out_lab_v, lab_hbm.at[b])
        pltpu.sync_copy(out_sco_v, sco_hbm.at[b])
        pltpu.sync_copy(boxof.at[pl.ds(0, K * 4)], box_hbm.at[b])
        pltpu.sync_copy(kpgf.at[pl.ds(0, K * 34)], kpt_hbm.at[b])


@jax.jit
def _run(lg, boxes2d, kpts2d, sizes_pad):
    mesh = plsc.VectorSubcoreMesh(core_axis_name="c", subcore_axis_name="s")
    fn = pl.kernel(
        _body,
        out_type=[
            jax.ShapeDtypeStruct((BATCH, SELP), jnp.int32),
            jax.ShapeDtypeStruct((BATCH, K * 4), jnp.float32),
            jax.ShapeDtypeStruct((BATCH, SELP), jnp.float32),
            jax.ShapeDtypeStruct((BATCH, K * 34), jnp.float32),
        ],
        mesh=mesh,
        compiler_params=pltpu.CompilerParams(needs_layout_passes=False,
                                             use_tc_tiling_on_sc=False),
        scratch_types=[
            pltpu.VMEM((2 * W,), jnp.float32),
            pltpu.VMEM((BINS * 16,), jnp.int32),
            pltpu.VMEM((BINS,), jnp.int32),
            pltpu.VMEM((GTCAP,), jnp.int32),
            pltpu.VMEM((GTCAP,), jnp.int32),
            pltpu.VMEM((CEQ + 16,), jnp.int32),
            pltpu.VMEM((CEQ + 16,), jnp.int32),
            pltpu.VMEM((SELP + 16,), jnp.int32),
            pltpu.VMEM((SELP,), jnp.int32),
            pltpu.VMEM((SELP,), jnp.int32),
            pltpu.VMEM((SELP,), jnp.int32),
            pltpu.VMEM((SELP,), jnp.float32),
            pltpu.VMEM((SELP,), jnp.int32),
            pltpu.VMEM((SELP,), jnp.int32),
            pltpu.VMEM((SELP * 4,), jnp.int32),
            pltpu.VMEM((SELP * 34,), jnp.int32),
            pltpu.VMEM((SELP * 4,), jnp.float32),
            pltpu.VMEM((SELP * 4,), jnp.float32),
            pltpu.VMEM((SELP * 34,), jnp.float32),
            pltpu.VMEM((16,), jnp.float32),
            pltpu.SemaphoreType.DMA,
            pltpu.SemaphoreType.DMA,
            pltpu.SemaphoreType.DMA,
            pltpu.SemaphoreType.DMA,
        ],
    )
    return fn(lg, boxes2d, kpts2d, sizes_pad)


def kernel(pred_logits, pred_boxes, pred_keypoints, orig_target_sizes):
    zero = lax.optimization_barrier(jnp.float32(0.0))
    lg = pred_logits.reshape(-1) + zero
    boxes1d = pred_boxes.reshape(-1) + zero
    kpts1d = pred_keypoints.reshape(-1) + zero
    sizes_pad = jnp.pad(orig_target_sizes.astype(jnp.float32),
                        ((0, 0), (0, 14)))
    lab_p, box_p, sco_p, kpt_p = _run(lg, boxes1d, kpts1d, sizes_pad)
    labels = lab_p[:, :K]
    scores = sco_p[:, :K]
    boxes = box_p.reshape(BATCH, K, 4)
    kpts = kpt_p.reshape(BATCH, K, 17, 2)
    return (labels, boxes, scores, kpts)


# speedup vs baseline: 1.5531x; 1.0004x over previous
import functools

import jax
import jax.numpy as jnp
from jax import lax
from jax.experimental import pallas as pl
from jax.experimental.pallas import tpu as pltpu
from jax.experimental.pallas import tpu_sc as plsc

NCLS = 80
NQ = 5000
K = 300
BATCH = 16
NFLAT = NQ * NCLS
W = 16000
NWIN = NFLAT // W
CHW = W // 16
BINS = 2048
CEQ = 4096
GTCAP = 640
SELP = 304
MASK31 = 0x7FFFFFFF


def _i32(x):
    return jnp.int32(x)


def _shrl(v, n):
    n_arr = jnp.broadcast_to(jnp.asarray(n, jnp.int32), jnp.shape(v))
    return lax.shift_right_logical(v, n_arr)


def _body(lg, boxes1d, kpts1d, sizes_hbm,
          lab_hbm, box_hbm, sco_hbm, kpt_hbm,
          wbuf, hist, totals,
          gt_key, gt_idx, eq_key, eq_idx, eq2_idx,
          sel_key, sel_idx,
          out_lab_v, out_sco_v, qrank_v, qabs_v,
          bidx_v, kidx_v, boxgf, boxof, kpgf, sizes_v,
          sem0, sem1, semg, semg2):
    cid = lax.axis_index("c")
    b = lax.axis_index("s")
    lane = lax.iota(jnp.int32, 16)
    ones16 = jnp.ones((16,), jnp.int32)
    zeros16 = jnp.zeros((16,), jnp.int32)
    MIN = _i32(-2**31)
    lane_off = lane * BINS
    b_base = b * NFLAT

    def keyof(x16):
        bb = lax.bitcast_convert_type(x16, jnp.int32)
        return jnp.where(bb < 0, ~(bb & _i32(MASK31)), bb)

    def zero_hist():
        def zh(i, _):
            for u in range(8):
                hist[pl.ds((i * 8 + u) * 16, 16)] = zeros16
            return 0
        lax.fori_loop(0, BINS * 16 // 16 // 8, zh, 0)

    def merge_lanes():
        def ml(j, _):
            acc = zeros16
            for l in range(16):
                acc = acc + hist[pl.ds(l * BINS + j * 16, 16)]
            totals[pl.ds(j * 16, 16)] = acc
            return 0
        lax.fori_loop(0, BINS // 16, ml, 0)

    def lane_at(vec, l):
        return jnp.sum(jnp.where(lane == l, vec, 0))

    def boundary_search(rn):
        def bs(jj, c):
            found, T, G, Q, acc = c
            j = BINS // 16 - 1 - jj
            t = totals[pl.ds(j * 16, 16)]
            rt = jnp.flip(t, 0)
            cs = plsc.cumsum(rt)
            s_inc = cs + acc
            hitm = s_inc >= rn
            cnt = jnp.sum(hitm.astype(jnp.int32))
            hit = jnp.logical_and(found == 0, cnt > 0)
            lstar = 16 - cnt
            csl = lane_at(cs, lstar)
            rtl = lane_at(rt, lstar)
            T = jnp.where(hit, j * 16 + 15 - lstar, T)
            G = jnp.where(hit, acc + csl - rtl, G)
            Q = jnp.where(hit, rtl, Q)
            found = jnp.where(hit, 1, found)
            acc = acc + jnp.sum(t)
            return (found, T, G, Q, acc)
        init = (_i32(0), _i32(0), _i32(0), _i32(0), _i32(0))
        _, T, G, Q, _ = lax.fori_loop(0, BINS // 16, bs, init)
        return T, G, Q

    def stream_windows(process, carry):
        pltpu.async_copy(lg.at[pl.ds(b_base, W)], wbuf.at[pl.ds(0, W)], sem0)

        def outer(w2, c):
            wA = 2 * w2
            pltpu.async_copy(lg.at[pl.ds(b_base + (wA + 1) * W, W)],
                             wbuf.at[pl.ds(W, W)], sem1)
            pltpu.make_async_copy(lg.at[pl.ds(b_base + wA * W, W)],
                                  wbuf.at[pl.ds(0, W)], sem0).wait()
            c = process(0, wA, c)
            pltpu.async_copy(lg.at[pl.ds(b_base + (wA + 2) * W, W)],
                             wbuf.at[pl.ds(0, W)], sem0)
            pltpu.make_async_copy(lg.at[pl.ds(b_base + (wA + 1) * W, W)],
                                  wbuf.at[pl.ds(W, W)], sem1).wait()
            c = process(W, wA + 1, c)
            return c
        carry = lax.fori_loop(0, (NWIN - 1) // 2, outer, carry)
        pltpu.make_async_copy(lg.at[pl.ds(b_base + (NWIN - 1) * W, W)],
                              wbuf.at[pl.ds(0, W)], sem0).wait()
        carry = process(0, NWIN - 1, carry)
        return carry

    @pl.when(cid == 0)
    def _work():
        pltpu.sync_copy(sizes_hbm.at[pl.ds(b * 128, 128)], sizes_v)

        zero_hist()

        def hist_chunk(soff, w, c):
            def inner(i, _):
                for u in range(8):
                    x = wbuf[pl.ds(soff + (i * 8 + u) * 16, 16)]
                    ku = keyof(x) ^ MIN
                    dig = _shrl(ku, 21)
                    plsc.addupdate_scatter(hist, [lane_off + dig], ones16)
                return 0
            lax.fori_loop(0, CHW // 8, inner, 0)
            return c
        stream_windows(hist_chunk, 0)
        merge_lanes()
        T, G, Q = boundary_search(_i32(K))
        shift = _i32(21)
        pref = T
        rn = _i32(K) - G

        def fb_cond(st):
            shift, pref, rn, Q = st
            return jnp.logical_and(Q > CEQ, shift > 0)

        def fb_body(st):
            shift, pref, rn, Q = st
            new_w = jnp.minimum(_i32(11), shift)
            new_shift = shift - new_w
            wmask = (_i32(1) << new_w) - 1
            zero_hist()

            def fw(w, _):
                pltpu.sync_copy(lg.at[pl.ds(b_base + w * W, W)],
                                wbuf.at[pl.ds(0, W)])

                def inner(i, _):
                    x = wbuf[pl.ds(0 + i * 16, 16)]
                    ku = keyof(x) ^ MIN
                    pm = _shrl(ku, shift) == pref
                    dig = _shrl(ku, new_shift) & wmask
                    plsc.addupdate_scatter(hist, [lane_off + dig], ones16,
                                           mask=pm)
                    return 0
                lax.fori_loop(0, CHW, inner, 0)
                return 0
            lax.fori_loop(0, NWIN, fw, 0)
            merge_lanes()
            T, G, Q = boundary_search(rn)
            return ((new_shift, (pref << new_w) | T, rn - G, Q))

        shift, pref, rn, Q = lax.while_loop(
            fb_cond, fb_body, (shift, pref, rn, Q))

        pref_b = pref ^ MIN

        def compact_chunk(soff, w, c):
            def inner(i, c2):
                vals = []
                anym = None
                for u in range(4):
                    x = wbuf[pl.ds(soff + (i * 4 + u) * 16, 16)]
                    ks = keyof(x)
                    top = _shrl(ks ^ MIN, shift)
                    m_gt = (top ^ MIN) > pref_b
                    m_eq = top == pref
                    vals.append((ks, m_gt, m_eq))
                    m_any = jnp.logical_or(m_gt, m_eq)
                    anym = m_any if anym is None else jnp.logical_or(anym,
                                                                     m_any)
                hit = jnp.sum(anym.astype(jnp.int32))

                def slow(c2):
                    off_gt, off_eq = c2
                    for u, (ks, m_gt, m_eq) in enumerate(vals):
                        fidx = w * W + (i * 4 + u) * 16 + lane
                        n_gt = jnp.sum(m_gt.astype(jnp.int32))
                        n_eq = jnp.sum(m_eq.astype(jnp.int32))
                        plsc.store_compressed(gt_key.at[pl.ds(off_gt, 16)],
                                              ks, mask=m_gt)
                        plsc.store_compressed(gt_idx.at[pl.ds(off_gt, 16)],
                                              fidx, mask=m_gt)

                        @pl.when(off_eq < CEQ)
                        def _se(ks=ks, m_eq=m_eq, fidx=fidx, off_eq=off_eq):
                            plsc.store_compressed(
                                eq_key.at[pl.ds(off_eq, 16)], ks, mask=m_eq)
                            plsc.store_compressed(
                                eq_idx.at[pl.ds(off_eq, 16)], fidx, mask=m_eq)
                        off_gt = off_gt + n_gt
                        off_eq = jnp.where(off_eq < CEQ, off_eq + n_eq,
                                           off_eq)
                    return (off_gt, off_eq)

                return lax.cond(hit > 0, slow, lambda c2: c2, c2)

            return lax.fori_loop(0, CHW // 4, inner, c)

        off_gt, off_eq = stream_windows(compact_chunk, (_i32(0), _i32(0)))
        Qs = off_eq
        nch_eq = (Qs + 15) // 16

        def local_hist(shift_p, pref_p, new_w, new_shift):
            zero_hist()
            wmask = (_i32(1) << new_w) - 1

            def lh(ci, _):
                kk = eq_key[pl.ds(ci * 16, 16)]
                valid = (ci * 16 + lane) < Qs
                ku2 = kk ^ MIN
                pm = jnp.logical_and(valid, _shrl(ku2, shift_p) == pref_p)
                dig = _shrl(ku2, new_shift) & wmask
                plsc.addupdate_scatter(hist, [lane_off + dig], ones16,
                                       mask=pm)
                return 0
            lax.fori_loop(0, nch_eq, lh, 0)
            merge_lanes()

        w1 = jnp.minimum(_i32(11), shift)
        sh1 = shift - w1
        local_hist(shift, pref, w1, sh1)
        T, G, Q = boundary_search(rn)
        pref = (pref << w1) | T
        rn = rn - G
        shift = sh1

        w2 = shift
        local_hist(shift, pref, w2, _i32(0))
        T, G, Q = boundary_search(rn)
        pref = (pref << w2) | T
        rn = rn - G
        E = rn
        k300_ks = pref ^ MIN

        def lsel(ci, c2):
            off_g, off_e2 = c2
            kk = eq_key[pl.ds(ci * 16, 16)]
            ii = eq_idx[pl.ds(ci * 16, 16)]
            valid = (ci * 16 + lane) < Qs
            m_g2 = jnp.logical_and(valid, kk > k300_ks)
            m_e2 = jnp.logical_and(valid, kk == k300_ks)
            n_g2 = jnp.sum(m_g2.astype(jnp.int32))
            n_e2 = jnp.sum(m_e2.astype(jnp.int32))

            @pl.when(n_g2 + n_e2 > 0)
            def _st():
                plsc.store_compressed(gt_key.at[pl.ds(off_g, 16)], kk,
                                      mask=m_g2)
                plsc.store_compressed(gt_idx.at[pl.ds(off_g, 16)], ii,
                                      mask=m_g2)

                @pl.when(off_e2 < SELP)
                def _se():
                    plsc.store_compressed(eq2_idx.at[pl.ds(off_e2, 16)], ii,
                                          mask=m_e2)
            off_g = off_g + n_g2
            off_e2 = jnp.where(off_e2 < SELP, off_e2 + n_e2, off_e2)
            return (off_g, off_e2)

        gp, _ = lax.fori_loop(0, nch_eq, lsel, (off_gt, _i32(0)))

        for c in range(SELP // 16):
            pos = c * 16 + lane
            in_gt = pos < gp
            gk = plsc.load_gather(gt_key, [jnp.minimum(pos, GTCAP - 1)],
                                  mask=in_gt)
            gi = plsc.load_gather(gt_idx, [jnp.minimum(pos, GTCAP - 1)],
                                  mask=in_gt)
            in_eq = jnp.logical_and(jnp.logical_not(in_gt), pos < K)
            ep = jnp.clip(pos - gp, 0, SELP + 15)
            ei = plsc.load_gather(eq2_idx, [ep], mask=in_eq)
            keyc = jnp.where(in_gt, gk, jnp.where(in_eq, k300_ks, MIN))
            idxc = jnp.where(in_gt, gi,
                             jnp.where(in_eq, ei, _i32(MASK31)))
            sel_key[pl.ds(c * 16, 16)] = keyc
            sel_idx[pl.ds(c * 16, 16)] = idxc

        def rank_j(j, ranks):
            jc = (j // 16) * 16
            lj = j - jc
            kv = sel_key[pl.ds(jc, 16)]
            iv = sel_idx[pl.ds(jc, 16)]
            kj = lane_at(kv, lj)
            ij = lane_at(iv, lj)
            new = []
            for c in range(SELP // 16):
                ki = sel_key[pl.ds(c * 16, 16)]
                ii2 = sel_idx[pl.ds(c * 16, 16)]
                beat = jnp.logical_or(
                    kj > ki,
                    jnp.logical_and(kj == ki, ij < ii2))
                new.append(ranks[c] + beat.astype(jnp.int32))
            return tuple(new)

        ranks0 = tuple(jnp.zeros((16,), jnp.int32)
                       for _ in range(SELP // 16))
        ranks = lax.fori_loop(0, SELP, rank_j, ranks0)

        out_lab_v[pl.ds(288, 16)] = zeros16
        out_sco_v[pl.ds(288, 16)] = jnp.zeros((16,), jnp.float32)
        qrank_v[pl.ds(288, 16)] = zeros16
        for c in range(SELP // 16):
            rk = ranks[c]
            valid = rk < K
            ksv = sel_key[pl.ds(c * 16, 16)]
            bb = jnp.where(ksv >= 0, ksv, (~ksv) | MIN)
            val = lax.bitcast_convert_type(bb, jnp.float32)
            score = 1.0 / (1.0 + jnp.exp(-val))
            fi = sel_idx[pl.ds(c * 16, 16)]
            q = fi // NCLS
            labv = fi - q * NCLS
            plsc.store_scatter(out_sco_v, [rk], score, mask=valid)
            plsc.store_scatter(out_lab_v, [rk], labv, mask=valid)
            plsc.store_scatter(qrank_v, [rk], q, mask=valid)

        for c in range(SELP // 16):
            qv = qrank_v[pl.ds(c * 16, 16)]
            qabs_v[pl.ds(c * 16, 16)] = qv + b * NQ

        for c in range(SELP * 4 // 16):
            rr = c * 4 + lax.shift_right_logical(lane, jnp.full((16,), 2,
                                                                jnp.int32))
            qv4 = plsc.load_gather(qabs_v, [rr])
            bidx_v[pl.ds(c * 16, 16)] = qv4 * 4 + (lane & 3)

        def kidx(ci, _):
            flat = ci * 16 + lane
            r = flat // 34
            col = flat - r * 34
            qv = plsc.load_gather(qabs_v, [r])
            kidx_v[pl.ds(ci * 16, 16)] = qv * 34 + col
            return 0
        lax.fori_loop(0, SELP * 34 // 16, kidx, 0)

        cb = pltpu.async_copy(boxes1d.at[bidx_v], boxgf, semg)
        ck = pltpu.async_copy(kpts1d.at[kidx_v], kpgf, semg2)
        cb.wait()
        ck.wait()

        sz = sizes_v[pl.ds(0, 16)]
        s0 = jnp.sum(jnp.where(lane == 0, sz, 0.0))
        s1 = jnp.sum(jnp.where(lane == 1, sz, 0.0))

        for c in range(SELP // 16):
            r4 = (c * 16 + lane) * 4
            cx = plsc.load_gather(boxgf, [r4])
            cy = plsc.load_gather(boxgf, [r4 + 1])
            wv = plsc.load_gather(boxgf, [r4 + 2])
            hv = plsc.load_gather(boxgf, [r4 + 3])
            hw = wv * 0.5
            hh = hv * 0.5
            plsc.store_scatter(boxof, [r4], (cx - hw) * s0)
            plsc.store_scatter(boxof, [r4 + 1], (cy - hh) * s1)
            plsc.store_scatter(boxof, [r4 + 2], (cx + hw) * s0)
            plsc.store_scatter(boxof, [r4 + 3], (cy + hh) * s1)

        whk = jnp.where((lane & 1) == 0, s0, s1)

        def kp(ci, _):
            kpgf[pl.ds(ci * 16, 16)] = kpgf[pl.ds(ci * 16, 16)] * whk
            return 0
        lax.fori_loop(0, SELP * 34 // 16, kp, 0)

        pltpu.sync_copy(out_lab_v, lab_hbm.at[pl.ds(b * 384, 384)])
        pltpu.sync_copy(out_sco_v, sco_hbm.at[pl.ds(b * 384, 384)])
        pltpu.sync_copy(boxof.at[pl.ds(0, 1280)],
                        box_hbm.at[pl.ds(b * 1280, 1280)])
        pltpu.sync_copy(kpgf.at[pl.ds(0, 10240)],
                        kpt_hbm.at[pl.ds(b * 10240, 10240)])


@jax.jit
def _run(lg, boxes2d, kpts2d, sizes_pad):
    mesh = plsc.VectorSubcoreMesh(core_axis_name="c", subcore_axis_name="s")
    fn = pl.kernel(
        _body,
        out_type=[
            jax.ShapeDtypeStruct((BATCH * 384,), jnp.int32),
            jax.ShapeDtypeStruct((BATCH * 1280,), jnp.float32),
            jax.ShapeDtypeStruct((BATCH * 384,), jnp.float32),
            jax.ShapeDtypeStruct((BATCH * 10240,), jnp.float32),
        ],
        mesh=mesh,
        compiler_params=pltpu.CompilerParams(needs_layout_passes=False),
        scratch_types=[
            pltpu.VMEM((2 * W,), jnp.float32),
            pltpu.VMEM((BINS * 16,), jnp.int32),
            pltpu.VMEM((BINS,), jnp.int32),
            pltpu.VMEM((GTCAP,), jnp.int32),
            pltpu.VMEM((GTCAP,), jnp.int32),
            pltpu.VMEM((CEQ + 16,), jnp.int32),
            pltpu.VMEM((CEQ + 16,), jnp.int32),
            pltpu.VMEM((SELP + 16,), jnp.int32),
            pltpu.VMEM((SELP,), jnp.int32),
            pltpu.VMEM((SELP,), jnp.int32),
            pltpu.VMEM((384,), jnp.int32),
            pltpu.VMEM((384,), jnp.float32),
            pltpu.VMEM((SELP,), jnp.int32),
            pltpu.VMEM((SELP,), jnp.int32),
            pltpu.VMEM((SELP * 4,), jnp.int32),
            pltpu.VMEM((SELP * 34,), jnp.int32),
            pltpu.VMEM((SELP * 4,), jnp.float32),
            pltpu.VMEM((1280,), jnp.float32),
            pltpu.VMEM((SELP * 34,), jnp.float32),
            pltpu.VMEM((128,), jnp.float32),
            pltpu.SemaphoreType.DMA,
            pltpu.SemaphoreType.DMA,
            pltpu.SemaphoreType.DMA,
            pltpu.SemaphoreType.DMA,
        ],
    )
    return fn(lg, boxes2d, kpts2d, sizes_pad)


def kernel(pred_logits, pred_boxes, pred_keypoints, orig_target_sizes):
    zero = lax.optimization_barrier(jnp.float32(0.0))
    lg = pred_logits.reshape(-1) + zero
    boxes1d = pred_boxes.reshape(-1) + zero
    kpts1d = pred_keypoints.reshape(-1) + zero
    sizes_pad = (jnp.pad(orig_target_sizes.astype(jnp.float32),
                         ((0, 0), (0, 126))).reshape(-1) + zero)
    lab_p, box_p, sco_p, kpt_p = _run(lg, boxes1d, kpts1d, sizes_pad)
    labels = lab_p.reshape(BATCH, 384)[:, :K]
    scores = sco_p.reshape(BATCH, 384)[:, :K]
    boxes = box_p.reshape(BATCH, 1280)[:, :K * 4].reshape(BATCH, K, 4)
    kpts = (kpt_p.reshape(BATCH, 10240)[:, :K * 34]
            .reshape(BATCH, K, 17, 2))
    return (labels, boxes, scores, kpts)
